# baseline (speedup 1.0000x reference)
import jax
import jax.numpy as jnp
from jax import lax
from jax.experimental import pallas as pl
from jax.experimental.pallas import tpu as pltpu
from jax.experimental.pallas import tpu_sc as plsc

_B = 16384
_D = 64
_L = 16
_NC = 2
_NS = 16
_NW = _NC * _NS
_BPW = _B // _NW
_CHUNK = 128
_NCHUNK = _BPW // _CHUNK


def _bpr_body(users_hbm, items_hbm, uemb_hbm, iemb_hbm, out_hbm,
              uidx_v, iidx_v, urows_v, irows_v, out_v, sem):
    wid = lax.axis_index("s") * _NC + lax.axis_index("c")
    base = wid * _BPW

    pltpu.sync_copy(users_hbm.at[wid], uidx_v)
    pltpu.sync_copy(items_hbm.at[wid], iidx_v)

    copies = []
    for j in range(_NCHUNK):
        copies.append(pltpu.async_copy(
            uemb_hbm.at[uidx_v.at[j]],
            urows_v.at[pl.ds(j * _CHUNK, _CHUNK)], sem))
        copies.append(pltpu.async_copy(
            iemb_hbm.at[iidx_v.at[j]],
            irows_v.at[pl.ds(j * _CHUNK, _CHUNK)], sem))
    for c in copies:
        c.wait()

    lane = lax.iota(jnp.int32, _L)
    rot_idx = [jnp.bitwise_and(lane + sh, _L - 1) for sh in (8, 4, 2, 1)]
    gat_dnums = lax.GatherDimensionNumbers(
        offset_dims=(), collapsed_slice_dims=(0,), start_index_map=(0,))

    def _lane_rotate(p, idx):
        return lax.gather(p, idx[:, None], gat_dnums, (1,),
                          mode=lax.GatherScatterMode.PROMISE_IN_BOUNDS)

    def group(g, carry):
        dots = jnp.zeros((_L,), jnp.float32)
        for k in range(_L):
            r = g * _L + k
            p = urows_v[r, pl.ds(0, _L)] * irows_v[r, pl.ds(0, _L)]
            for c in range(1, _D // _L):
                p = p + urows_v[r, pl.ds(c * _L, _L)] * irows_v[r, pl.ds(c * _L, _L)]
            for idx in rot_idx:
                p = p + _lane_rotate(p, idx)
            dots = jnp.where(lane == k, p, dots)
        out_v[pl.ds(g * _L, _L)] = dots
        return carry

    lax.fori_loop(0, _BPW // _L, group, 0)
    pltpu.sync_copy(out_v, out_hbm.at[pl.ds(base, _BPW)])


def kernel(users, items, user_emb, item_emb):
    users3 = users.astype(jnp.int32).reshape(_NW, _NCHUNK, _CHUNK)
    items3 = items.astype(jnp.int32).reshape(_NW, _NCHUNK, _CHUNK)
    mesh = plsc.VectorSubcoreMesh(core_axis_name="c", subcore_axis_name="s")
    run = pl.kernel(
        _bpr_body,
        out_type=jax.ShapeDtypeStruct((_B,), jnp.float32),
        mesh=mesh,
        compiler_params=pltpu.CompilerParams(use_tc_tiling_on_sc=False),
        scratch_types=[
            pltpu.VMEM((_NCHUNK, _CHUNK), jnp.int32),
            pltpu.VMEM((_NCHUNK, _CHUNK), jnp.int32),
            pltpu.VMEM((_BPW, _D), jnp.float32),
            pltpu.VMEM((_BPW, _D), jnp.float32),
            pltpu.VMEM((_BPW,), jnp.float32),
            pltpu.SemaphoreType.DMA,
        ],
    )
    return run(users3, items3, user_emb, item_emb)


# speedup vs baseline: 2.1394x; 2.1394x over previous
import jax
import jax.numpy as jnp
from jax import lax
from jax.experimental import pallas as pl
from jax.experimental.pallas import tpu as pltpu
from jax.experimental.pallas import tpu_sc as plsc

_B = 16384
_D = 64
_L = 16
_NC = 2
_NS = 16
_NW = _NC * _NS
_BPW = _B // _NW
_NG = _BPW // _L
_NBLK = 125000


def _bpr_body(utid_hbm, itid_hbm, usub_hbm, isub_hbm, uemb_hbm, iemb_hbm,
              out_hbm, utid_v, itid_v, usub_v, isub_v, ublk_v, iblk_v,
              out_v, sem):
    wid = lax.axis_index("s") * _NC + lax.axis_index("c")
    base = wid * _BPW

    pltpu.sync_copy(utid_hbm.at[wid], utid_v)
    pltpu.sync_copy(itid_hbm.at[wid], itid_v)
    pltpu.sync_copy(usub_hbm.at[wid], usub_v)
    pltpu.sync_copy(isub_hbm.at[wid], isub_v)

    lane = lax.iota(jnp.int32, _L)
    gat_dnums = lax.GatherDimensionNumbers(
        offset_dims=(), collapsed_slice_dims=(0,), start_index_map=(0,))
    rot_idx = [jnp.bitwise_and(lane + sh, _L - 1) for sh in (8, 4, 2, 1)]

    def _lane_rotate(p, idx):
        return lax.gather(p, idx[:, None], gat_dnums, (1,),
                          mode=lax.GatherScatterMode.PROMISE_IN_BOUNDS)

    def group(g, carry):
        utv = utid_v[pl.ds(g * _L, _L)]
        itv = itid_v[pl.ds(g * _L, _L)]
        for k in range(_L):
            pltpu.async_copy(uemb_hbm.at[utv[k]], ublk_v.at[k], sem)
            pltpu.async_copy(iemb_hbm.at[itv[k]], iblk_v.at[k], sem)
        pltpu.make_async_copy(uemb_hbm.at[pl.ds(0, _L)], ublk_v, sem).wait()
        pltpu.make_async_copy(iemb_hbm.at[pl.ds(0, _L)], iblk_v, sem).wait()

        suv = usub_v[pl.ds(g * _L, _L)]
        siv = isub_v[pl.ds(g * _L, _L)]
        dots = jnp.zeros((_L,), jnp.float32)
        for k in range(_L):
            su = suv[k]
            si = siv[k]
            p = ublk_v[k, su, pl.ds(0, _L)] * iblk_v[k, si, pl.ds(0, _L)]
            for c in range(1, _D // _L):
                p = p + (ublk_v[k, su, pl.ds(c * _L, _L)]
                         * iblk_v[k, si, pl.ds(c * _L, _L)])
            for idx in rot_idx:
                p = p + _lane_rotate(p, idx)
            dots = jnp.where(lane == k, p, dots)
        out_v[pl.ds(g * _L, _L)] = dots
        return carry

    lax.fori_loop(0, _NG, group, 0)
    pltpu.sync_copy(out_v, out_hbm.at[pl.ds(base, _BPW)])


def kernel(users, items, user_emb, item_emb):
    users = users.astype(jnp.int32)
    items = items.astype(jnp.int32)
    utid = (users >> 3).reshape(_NW, _BPW)
    itid = (items >> 3).reshape(_NW, _BPW)
    usub = (users & 7).reshape(_NW, _BPW)
    isub = (items & 7).reshape(_NW, _BPW)
    uemb3 = user_emb.reshape(_NBLK, 8, _D)
    iemb3 = item_emb.reshape(_NBLK, 8, _D)
    mesh = plsc.VectorSubcoreMesh(core_axis_name="c", subcore_axis_name="s")
    run = pl.kernel(
        _bpr_body,
        out_type=jax.ShapeDtypeStruct((_B,), jnp.float32),
        mesh=mesh,
        scratch_types=[
            pltpu.VMEM((_BPW,), jnp.int32),
            pltpu.VMEM((_BPW,), jnp.int32),
            pltpu.VMEM((_BPW,), jnp.int32),
            pltpu.VMEM((_BPW,), jnp.int32),
            pltpu.VMEM((_L, 8, _D), jnp.float32),
            pltpu.VMEM((_L, 8, _D), jnp.float32),
            pltpu.VMEM((_BPW,), jnp.float32),
            pltpu.SemaphoreType.DMA,
        ],
    )
    return run(utid, itid, usub, isub, uemb3, iemb3)
